# baseline (speedup 1.0000x reference)
import functools

import jax
import jax.numpy as jnp
from jax import lax
from jax.experimental import pallas as pl
from jax.experimental.pallas import tpu as pltpu
from jax.experimental.pallas import tpu_sc as plsc

NC = 2
NS = 16
CHUNK = 128


def _weights_xw(x, Wp, bp, wv, bv):
    N, D = x.shape
    H = Wp.shape[1]
    BLK = 1000

    def body(x_ref, wp_ref, bp_ref, wv_ref, bv_ref, o_ref):
        xb = x_ref[...]
        h = jnp.tanh(jnp.dot(xb, wp_ref[...], preferred_element_type=jnp.float32)
                     + bp_ref[...])
        w = jax.nn.sigmoid(jnp.dot(h, wv_ref[...], preferred_element_type=jnp.float32)
                           + bv_ref[...])
        o_ref[...] = xb * w

    return pl.pallas_call(
        body,
        grid=(N // BLK,),
        in_specs=[
            pl.BlockSpec((BLK, D), lambda i: (i, 0)),
            pl.BlockSpec((D, H), lambda i: (0, 0)),
            pl.BlockSpec((1, H), lambda i: (0, 0)),
            pl.BlockSpec((H, 1), lambda i: (0, 0)),
            pl.BlockSpec((1, 1), lambda i: (0, 0)),
        ],
        out_specs=pl.BlockSpec((BLK, D), lambda i: (i, 0)),
        out_shape=jax.ShapeDtypeStruct((N, D), jnp.float32),
    )(x, Wp, bp.reshape(1, H), wv, bv.reshape(1, 1))


def _scatter_sc(xw, colp, rowp, np_rows, cpw0, cpw1):
    n, D = xw.shape
    EP = colp.shape[0]
    rpt = np_rows // NS
    mesh = plsc.VectorSubcoreMesh(core_axis_name="c", subcore_axis_name="s")

    @functools.partial(
        pl.kernel,
        mesh=mesh,
        out_type=jax.ShapeDtypeStruct((NC, np_rows, D), jnp.float32),
        scratch_types=[
            pltpu.VMEM((CHUNK,), jnp.int32),
            pltpu.VMEM((CHUNK,), jnp.int32),
            pltpu.VMEM((CHUNK,), jnp.int32),
            pltpu.VMEM((CHUNK,), jnp.int32),
            pltpu.VMEM((CHUNK, D), jnp.float32),
            pltpu.VMEM((CHUNK, D), jnp.float32),
            pltpu.VMEM_SHARED((np_rows, D), jnp.float32),
            pltpu.SemaphoreType.DMA,
            pltpu.SemaphoreType.DMA,
        ],
    )
    def k(xw_hbm, col_hbm, row_hbm, out_hbm, cb0, cb1, rb0, rb1, g0, g1,
          acc, sem0, sem1):
        cbs, rbs, gbs, sems = (cb0, cb1), (rb0, rb1), (g0, g1), (sem0, sem1)
        c = lax.axis_index("c")
        s = lax.axis_index("s")

        def zrow(r, _):
            for v in range(D // 16):
                g0[r, pl.ds(v * 16, 16)] = jnp.zeros((16,), jnp.float32)
            return 0
        lax.fori_loop(0, CHUNK, zrow, 0)

        base = s * rpt
        off = 0
        rem = rpt
        while rem > 0:
            sz = min(CHUNK, rem)
            pltpu.sync_copy(g0.at[pl.ds(0, sz)], acc.at[pl.ds(base + off, sz)])
            off += sz
            rem -= sz
        plsc.subcore_barrier()

        def run_pipe(cpw, chunk0):
            def ifetch(kk, m):
                e0 = (chunk0 + kk) * CHUNK
                pltpu.sync_copy(col_hbm.at[pl.ds(e0, CHUNK)], cbs[m])
                pltpu.sync_copy(row_hbm.at[pl.ds(e0, CHUNK)], rbs[m])

            def gstart(m):
                pltpu.async_copy(xw_hbm.at[cbs[m]], gbs[m], sems[m])

            def gwait(m):
                pltpu.make_async_copy(xw_hbm.at[cbs[0]], gbs[m],
                                      sems[m]).wait()

            def step(kk, m, last=False):
                if not last:
                    ifetch(kk + 1, 1 - m)
                    gstart(1 - m)
                gwait(m)
                pltpu.sync_copy(gbs[m], acc.at[rbs[m]], add=True)

            ifetch(0, 0)
            gstart(0)

            def body(g, _):
                kk = 2 * g
                step(kk, 0)
                step(kk + 1, 1)
                return 0
            lax.fori_loop(0, (cpw - 2) // 2, body, 0)
            if cpw % 2 == 0:
                step(cpw - 2, 0)
                step(cpw - 1, 1, last=True)
            else:
                step(cpw - 1, (cpw - 1) % 2, last=True)

        @pl.when(c == 0)
        def _():
            run_pipe(cpw0, s * cpw0)

        @pl.when(c == 1)
        def _():
            run_pipe(cpw1, NS * cpw0 + s * cpw1)

        plsc.subcore_barrier()

        pltpu.sync_copy(acc.at[pl.ds(base, rpt)],
                        out_hbm.at[c, pl.ds(base, rpt)])

    return k(xw, colp, rowp)


def _prompt_out(parts, x, W1, b1, W2, b2):
    N, D = x.shape
    H = W1.shape[1]
    BLK = 1000

    def body(a0_ref, a1_ref, x_ref, w1_ref, b1_ref, w2_ref, b2_ref, o_ref):
        r = a0_ref[0] + a1_ref[0]
        t = jnp.maximum(jnp.dot(r, w1_ref[...], preferred_element_type=jnp.float32)
                        + b1_ref[...], 0.0)
        p = jnp.dot(t, w2_ref[...], preferred_element_type=jnp.float32) + b2_ref[...]
        o_ref[...] = x_ref[...] + p

    return pl.pallas_call(
        body,
        grid=(N // BLK,),
        in_specs=[
            pl.BlockSpec((1, BLK, D), lambda i: (0, i, 0)),
            pl.BlockSpec((1, BLK, D), lambda i: (1, i, 0)),
            pl.BlockSpec((BLK, D), lambda i: (i, 0)),
            pl.BlockSpec((D, H), lambda i: (0, 0)),
            pl.BlockSpec((1, H), lambda i: (0, 0)),
            pl.BlockSpec((H, D), lambda i: (0, 0)),
            pl.BlockSpec((1, D), lambda i: (0, 0)),
        ],
        out_specs=pl.BlockSpec((BLK, D), lambda i: (i, 0)),
        out_shape=jax.ShapeDtypeStruct((N, D), jnp.float32),
    )(parts, parts, x, W1, b1.reshape(1, H), W2, b2.reshape(1, D))


def kernel(x, edge_index, Wp, bp, wv, bv, W1, b1, W2, b2):
    N, D = x.shape
    E = edge_index.shape[1]
    gran = NC * NS * CHUNK
    EP = ((E + gran - 1) // gran) * gran
    np_rows = -(-(N + 1) // (NS * 8)) * (NS * 8)

    row = edge_index[0]
    col = edge_index[1]
    pad = EP - E
    if pad:
        rowp = jnp.concatenate(
            [row, N + (jnp.arange(pad, dtype=jnp.int32) % (np_rows - N))])
        colp = jnp.concatenate([col, jnp.zeros((pad,), jnp.int32)])
    else:
        rowp, colp = row, col

    nchunks = EP // CHUNK
    cpw0 = 52
    cpw1 = nchunks // NS - cpw0
    assert NS * (cpw0 + cpw1) == nchunks and cpw0 % 2 == 0 and cpw1 % 2 == 0

    xw = _weights_xw(x, Wp, bp, wv, bv)
    parts = _scatter_sc(xw, colp, rowp, np_rows, cpw0, cpw1)
    out = _prompt_out(parts, x, W1, b1, W2, b2)
    return (out, edge_index)


# speedup vs baseline: 1.2192x; 1.2192x over previous
import functools

import jax
import jax.numpy as jnp
from jax import lax
from jax.experimental import pallas as pl
from jax.experimental.pallas import tpu as pltpu
from jax.experimental.pallas import tpu_sc as plsc

NC = 2
NS = 16
CHUNK = 128


def _weights_xw(x, Wp, bp, wv, bv):
    N, D = x.shape
    H = Wp.shape[1]
    BLK = 1000

    def body(x_ref, wp_ref, bp_ref, wv_ref, bv_ref, o_ref):
        xb = x_ref[...]
        h = jnp.tanh(jnp.dot(xb, wp_ref[...], preferred_element_type=jnp.float32)
                     + bp_ref[...])
        w = jax.nn.sigmoid(jnp.dot(h, wv_ref[...], preferred_element_type=jnp.float32)
                           + bv_ref[...])
        o_ref[...] = xb * w

    return pl.pallas_call(
        body,
        grid=(N // BLK,),
        in_specs=[
            pl.BlockSpec((BLK, D), lambda i: (i, 0)),
            pl.BlockSpec((D, H), lambda i: (0, 0)),
            pl.BlockSpec((1, H), lambda i: (0, 0)),
            pl.BlockSpec((H, 1), lambda i: (0, 0)),
            pl.BlockSpec((1, 1), lambda i: (0, 0)),
        ],
        out_specs=pl.BlockSpec((BLK, D), lambda i: (i, 0)),
        out_shape=jax.ShapeDtypeStruct((N, D), jnp.float32),
    )(x, Wp, bp.reshape(1, H), wv, bv.reshape(1, 1))


def _scatter_sc(xw, colp, rowp, np_rows, cpw0, cpw1):
    n, D = xw.shape
    EP = colp.shape[0]
    rpt = np_rows // NS
    mesh = plsc.VectorSubcoreMesh(core_axis_name="c", subcore_axis_name="s")

    @functools.partial(
        pl.kernel,
        mesh=mesh,
        out_type=jax.ShapeDtypeStruct((NC, np_rows, D), jnp.float32),
        scratch_types=[
            pltpu.VMEM((CHUNK,), jnp.int32),
            pltpu.VMEM((CHUNK,), jnp.int32),
            pltpu.VMEM((CHUNK,), jnp.int32),
            pltpu.VMEM((CHUNK,), jnp.int32),
            pltpu.VMEM((CHUNK, D), jnp.float32),
            pltpu.VMEM((CHUNK, D), jnp.float32),
            pltpu.VMEM_SHARED((np_rows, D), jnp.float32),
            pltpu.SemaphoreType.DMA,
            pltpu.SemaphoreType.DMA,
        ],
    )
    def k(xw_hbm, col_hbm, row_hbm, out_hbm, cb0, cb1, rb0, rb1, g0, g1,
          acc, sem0, sem1):
        cbs, rbs, gbs, sems = (cb0, cb1), (rb0, rb1), (g0, g1), (sem0, sem1)
        c = lax.axis_index("c")
        s = lax.axis_index("s")

        def zrow(r, _):
            for v in range(D // 16):
                g0[r, pl.ds(v * 16, 16)] = jnp.zeros((16,), jnp.float32)
            return 0
        lax.fori_loop(0, CHUNK, zrow, 0)

        base = s * rpt
        off = 0
        rem = rpt
        while rem > 0:
            sz = min(CHUNK, rem)
            pltpu.sync_copy(g0.at[pl.ds(0, sz)], acc.at[pl.ds(base + off, sz)])
            off += sz
            rem -= sz
        plsc.subcore_barrier()

        def run_pipe(cpw, chunk0):
            def ifetch(kk, m):
                e0 = (chunk0 + kk) * CHUNK
                pltpu.sync_copy(col_hbm.at[pl.ds(e0, CHUNK)], cbs[m])
                pltpu.sync_copy(row_hbm.at[pl.ds(e0, CHUNK)], rbs[m])

            def gstart(m):
                pltpu.async_copy(xw_hbm.at[cbs[m]], gbs[m], sems[m])

            def gwait(m):
                pltpu.make_async_copy(xw_hbm.at[cbs[0]], gbs[m],
                                      sems[m]).wait()

            def step(kk, m, last=False):
                if not last:
                    ifetch(kk + 1, 1 - m)
                    gstart(1 - m)
                gwait(m)
                pltpu.sync_copy(gbs[m], acc.at[rbs[m]], add=True)

            ifetch(0, 0)
            gstart(0)

            def body(g, _):
                kk = 2 * g
                step(kk, 0)
                step(kk + 1, 1)
                return 0
            lax.fori_loop(0, (cpw - 2) // 2, body, 0)
            if cpw % 2 == 0:
                step(cpw - 2, 0)
                step(cpw - 1, 1, last=True)
            else:
                step(cpw - 1, (cpw - 1) % 2, last=True)

        @pl.when(c == 0)
        def _():
            run_pipe(cpw0, s * cpw0)

        @pl.when(c == 1)
        def _():
            run_pipe(cpw1, NS * cpw0 + s * cpw1)

        plsc.subcore_barrier()

        pltpu.sync_copy(acc.at[pl.ds(base, rpt)],
                        out_hbm.at[c, pl.ds(base, rpt)])

    return k(xw, colp, rowp)


def _prompt_out(parts, x, W1, b1, W2, b2):
    N, D = x.shape
    H = W1.shape[1]
    BLK = 1000

    def body(a0_ref, a1_ref, x_ref, w1_ref, b1_ref, w2_ref, b2_ref, o_ref):
        r = a0_ref[0] + a1_ref[0]
        t = jnp.maximum(jnp.dot(r, w1_ref[...], preferred_element_type=jnp.float32)
                        + b1_ref[...], 0.0)
        p = jnp.dot(t, w2_ref[...], preferred_element_type=jnp.float32) + b2_ref[...]
        o_ref[...] = x_ref[...] + p

    return pl.pallas_call(
        body,
        grid=(N // BLK,),
        in_specs=[
            pl.BlockSpec((1, BLK, D), lambda i: (0, i, 0)),
            pl.BlockSpec((1, BLK, D), lambda i: (1, i, 0)),
            pl.BlockSpec((BLK, D), lambda i: (i, 0)),
            pl.BlockSpec((D, H), lambda i: (0, 0)),
            pl.BlockSpec((1, H), lambda i: (0, 0)),
            pl.BlockSpec((H, D), lambda i: (0, 0)),
            pl.BlockSpec((1, D), lambda i: (0, 0)),
        ],
        out_specs=pl.BlockSpec((BLK, D), lambda i: (i, 0)),
        out_shape=jax.ShapeDtypeStruct((N, D), jnp.float32),
    )(parts, parts, x, W1, b1.reshape(1, H), W2, b2.reshape(1, D))


def kernel(x, edge_index, Wp, bp, wv, bv, W1, b1, W2, b2):
    N, D = x.shape
    E = edge_index.shape[1]
    gran = NC * NS * CHUNK
    EP = ((E + gran - 1) // gran) * gran
    np_rows = -(-(N + 1) // (NS * 8)) * (NS * 8)

    row = edge_index[0]
    col = edge_index[1]
    pad = EP - E
    if pad:
        rowp = jnp.concatenate(
            [row, N + (jnp.arange(pad, dtype=jnp.int32) % (np_rows - N))])
        colp = jnp.concatenate([col, jnp.zeros((pad,), jnp.int32)])
    else:
        rowp, colp = row, col

    nchunks = EP // CHUNK
    cpw0 = 106
    cpw1 = nchunks // NS - cpw0
    assert NS * (cpw0 + cpw1) == nchunks and cpw0 % 2 == 0 and cpw1 % 2 == 0

    xw = _weights_xw(x, Wp, bp, wv, bv)
    parts = _scatter_sc(xw, colp, rowp, np_rows, cpw0, cpw1)
    out = _prompt_out(parts, x, W1, b1, W2, b2)
    return (out, edge_index)


# speedup vs baseline: 1.2586x; 1.0323x over previous
import functools

import jax
import jax.numpy as jnp
from jax import lax
from jax.experimental import pallas as pl
from jax.experimental.pallas import tpu as pltpu
from jax.experimental.pallas import tpu_sc as plsc

NC = 2
NS = 16
CHUNK = 128


def _weights_xw(x, Wp, bp, wv, bv):
    N, D = x.shape
    H = Wp.shape[1]
    BLK = 1000

    def body(x_ref, wp_ref, bp_ref, wv_ref, bv_ref, o_ref):
        xb = x_ref[...]
        h = jnp.tanh(jnp.dot(xb, wp_ref[...], preferred_element_type=jnp.float32)
                     + bp_ref[...])
        w = jax.nn.sigmoid(jnp.dot(h, wv_ref[...], preferred_element_type=jnp.float32)
                           + bv_ref[...])
        o_ref[...] = xb * w

    return pl.pallas_call(
        body,
        grid=(N // BLK,),
        in_specs=[
            pl.BlockSpec((BLK, D), lambda i: (i, 0)),
            pl.BlockSpec((D, H), lambda i: (0, 0)),
            pl.BlockSpec((1, H), lambda i: (0, 0)),
            pl.BlockSpec((H, 1), lambda i: (0, 0)),
            pl.BlockSpec((1, 1), lambda i: (0, 0)),
        ],
        out_specs=pl.BlockSpec((BLK, D), lambda i: (i, 0)),
        out_shape=jax.ShapeDtypeStruct((N, D), jnp.float32),
    )(x, Wp, bp.reshape(1, H), wv, bv.reshape(1, 1))


def _scatter_sc(xw, colp, rowp, np_rows, cpw0, cpw1):
    n, D = xw.shape
    EP = colp.shape[0]
    rpt = np_rows // NS
    mesh = plsc.VectorSubcoreMesh(core_axis_name="c", subcore_axis_name="s")

    @functools.partial(
        pl.kernel,
        mesh=mesh,
        out_type=jax.ShapeDtypeStruct((NC, np_rows, D), jnp.float32),
        scratch_types=[
            pltpu.VMEM((CHUNK,), jnp.int32),
            pltpu.VMEM((CHUNK,), jnp.int32),
            pltpu.VMEM((CHUNK,), jnp.int32),
            pltpu.VMEM((CHUNK,), jnp.int32),
            pltpu.VMEM((CHUNK, D), jnp.float32),
            pltpu.VMEM((CHUNK, D), jnp.float32),
            pltpu.VMEM_SHARED((np_rows, D), jnp.float32),
            pltpu.SemaphoreType.DMA,
            pltpu.SemaphoreType.DMA,
        ],
    )
    def k(xw_hbm, col_hbm, row_hbm, out_hbm, cb0, cb1, rb0, rb1, g0, g1,
          acc, sem0, sem1):
        cbs, rbs, gbs, sems = (cb0, cb1), (rb0, rb1), (g0, g1), (sem0, sem1)
        c = lax.axis_index("c")
        s = lax.axis_index("s")

        def zrow(r, _):
            for v in range(D // 16):
                g0[r, pl.ds(v * 16, 16)] = jnp.zeros((16,), jnp.float32)
            return 0
        lax.fori_loop(0, CHUNK, zrow, 0)

        base = s * rpt
        off = 0
        rem = rpt
        while rem > 0:
            sz = min(CHUNK, rem)
            pltpu.sync_copy(g0.at[pl.ds(0, sz)], acc.at[pl.ds(base + off, sz)])
            off += sz
            rem -= sz
        plsc.subcore_barrier()

        def run_pipe(cpw, chunk0):
            def ifetch(kk, m):
                e0 = (chunk0 + kk) * CHUNK
                pltpu.sync_copy(col_hbm.at[pl.ds(e0, CHUNK)], cbs[m])
                pltpu.sync_copy(row_hbm.at[pl.ds(e0, CHUNK)], rbs[m])

            def gstart(m):
                pltpu.async_copy(xw_hbm.at[cbs[m]], gbs[m], sems[m])

            def gwait(m):
                pltpu.make_async_copy(xw_hbm.at[cbs[0]], gbs[m],
                                      sems[m]).wait()

            def step(kk, m, last=False):
                if not last:
                    ifetch(kk + 1, 1 - m)
                    gstart(1 - m)
                gwait(m)
                pltpu.sync_copy(gbs[m], acc.at[rbs[m]], add=True)

            ifetch(0, 0)
            gstart(0)

            def body(g, _):
                kk = 2 * g
                step(kk, 0)
                step(kk + 1, 1)
                return 0
            lax.fori_loop(0, (cpw - 2) // 2, body, 0)
            if cpw % 2 == 0:
                step(cpw - 2, 0)
                step(cpw - 1, 1, last=True)
            else:
                step(cpw - 1, (cpw - 1) % 2, last=True)

        @pl.when(c == 0)
        def _():
            run_pipe(cpw0, s * cpw0)

        @pl.when(c == 1)
        def _():
            run_pipe(cpw1, NS * cpw0 + s * cpw1)

        plsc.subcore_barrier()

        pltpu.sync_copy(acc.at[pl.ds(base, rpt)],
                        out_hbm.at[c, pl.ds(base, rpt)])

    return k(xw, colp, rowp)


def _prompt_out(parts, x, W1, b1, W2, b2):
    N, D = x.shape
    H = W1.shape[1]
    BLK = 1000

    def body(a0_ref, a1_ref, x_ref, w1_ref, b1_ref, w2_ref, b2_ref, o_ref):
        r = a0_ref[0] + a1_ref[0]
        t = jnp.maximum(jnp.dot(r, w1_ref[...], preferred_element_type=jnp.float32)
                        + b1_ref[...], 0.0)
        p = jnp.dot(t, w2_ref[...], preferred_element_type=jnp.float32) + b2_ref[...]
        o_ref[...] = x_ref[...] + p

    return pl.pallas_call(
        body,
        grid=(N // BLK,),
        in_specs=[
            pl.BlockSpec((1, BLK, D), lambda i: (0, i, 0)),
            pl.BlockSpec((1, BLK, D), lambda i: (1, i, 0)),
            pl.BlockSpec((BLK, D), lambda i: (i, 0)),
            pl.BlockSpec((D, H), lambda i: (0, 0)),
            pl.BlockSpec((1, H), lambda i: (0, 0)),
            pl.BlockSpec((H, D), lambda i: (0, 0)),
            pl.BlockSpec((1, D), lambda i: (0, 0)),
        ],
        out_specs=pl.BlockSpec((BLK, D), lambda i: (i, 0)),
        out_shape=jax.ShapeDtypeStruct((N, D), jnp.float32),
    )(parts, parts, x, W1, b1.reshape(1, H), W2, b2.reshape(1, D))


def kernel(x, edge_index, Wp, bp, wv, bv, W1, b1, W2, b2):
    N, D = x.shape
    E = edge_index.shape[1]
    gran = NC * NS * CHUNK
    EP = ((E + gran - 1) // gran) * gran
    np_rows = -(-(N + 1) // (NS * 8)) * (NS * 8)

    row = edge_index[0]
    col = edge_index[1]
    pad = EP - E
    if pad:
        rowp = jnp.concatenate(
            [row, N + (jnp.arange(pad, dtype=jnp.int32) % (np_rows - N))])
        colp = jnp.concatenate([col, jnp.zeros((pad,), jnp.int32)])
    else:
        rowp, colp = row, col

    nchunks = EP // CHUNK
    cpw0 = 114
    cpw1 = nchunks // NS - cpw0
    assert NS * (cpw0 + cpw1) == nchunks and cpw0 % 2 == 0 and cpw1 % 2 == 0

    xw = _weights_xw(x, Wp, bp, wv, bv)
    parts = _scatter_sc(xw, colp, rowp, np_rows, cpw0, cpw1)
    out = _prompt_out(parts, x, W1, b1, W2, b2)
    return (out, edge_index)
